# baseline (speedup 1.0000x reference)
import functools

import jax
import jax.numpy as jnp
from jax import lax
from jax.experimental import pallas as pl
from jax.experimental.pallas import tpu as pltpu
from jax.experimental.pallas import tpu_sc as plsc

SEG = 1024
NC = 2
NS = 16
NW = NC * NS


def _score_body(x_ref, w1_ref, b1_ref, w2_ref, w2r_ref, b2_ref,
                wx_ref, e_ref):
    x = x_ref[...]
    h = jnp.tanh(
        jax.lax.dot_general(x, w1_ref[...], (((1,), (0,)), ((), ())),
                            preferred_element_type=jnp.float32)
        + b1_ref[...])
    s = jax.lax.dot_general(h, w2_ref[...], (((1,), (0,)), ((), ())),
                            preferred_element_type=jnp.float32) + b2_ref[...]
    wx_ref[...] = x * jnp.exp(s)
    s_row = jax.lax.dot_general(w2r_ref[...], h, (((1,), (1,)), ((), ())),
                                preferred_element_type=jnp.float32)
    e_ref[...] = jnp.exp(s_row + b2_ref[...])[None]


def _scores_premul(x, w1, b1, w2, b2, block_rows, row_start=0,
                   row_count=None, interpret=False):
    n, d = x.shape
    nh = n if row_count is None else row_count
    grid = nh // block_rows
    blk0 = row_start // block_rows
    wx, e = pl.pallas_call(
        _score_body,
        grid=(grid,),
        in_specs=[
            pl.BlockSpec((block_rows, d), lambda i: (i + blk0, 0)),
            pl.BlockSpec((d, w1.shape[1]), lambda i: (0, 0)),
            pl.BlockSpec((1, w1.shape[1]), lambda i: (0, 0)),
            pl.BlockSpec((w1.shape[1], 1), lambda i: (0, 0)),
            pl.BlockSpec((1, w1.shape[1]), lambda i: (0, 0)),
            pl.BlockSpec((1, 1), lambda i: (0, 0)),
        ],
        out_specs=[
            pl.BlockSpec((block_rows, d), lambda i: (i, 0)),
            pl.BlockSpec((1, 1, block_rows), lambda i: (i, 0, 0)),
        ],
        out_shape=[
            jax.ShapeDtypeStruct((nh, d), jnp.float32),
            jax.ShapeDtypeStruct((grid, 1, block_rows), jnp.float32),
        ],
        interpret=interpret,
    )(x, w1, b1.reshape(1, -1), w2, w2.reshape(1, -1), b2.reshape(1, 1))
    return wx, e.reshape(nh)


def _sc_scatter_call(wx, e, batch, chunk):
    n, d = wx.shape
    rows_per_w = n // NW
    n_chunks = rows_per_w // chunk
    mesh = plsc.VectorSubcoreMesh(core_axis_name="c", subcore_axis_name="s")
    seg_per_sub = SEG // NS
    assert n_chunks % 2 == 1

    @functools.partial(
        pl.kernel,
        out_type=[
            jax.ShapeDtypeStruct((NC, SEG, d), jnp.float32),
            jax.ShapeDtypeStruct((NC, SEG), jnp.float32),
        ],
        mesh=mesh,
        scratch_types=[
            pltpu.VMEM((chunk, d), jnp.float32),
            pltpu.VMEM((chunk, d), jnp.float32),
            pltpu.VMEM((chunk,), jnp.float32),
            pltpu.VMEM((chunk,), jnp.float32),
            pltpu.VMEM((chunk,), jnp.int32),
            pltpu.VMEM((chunk,), jnp.int32),
            pltpu.VMEM_SHARED((SEG, d), jnp.float32),
            pltpu.VMEM_SHARED((SEG,), jnp.float32),
            pltpu.SemaphoreType.DMA,
            pltpu.SemaphoreType.DMA,
        ],
    )
    def sc_kernel(wx_hbm, e_hbm, batch_hbm, acc_hbm, den_hbm,
                  rows0, rows1, e0, e1, idx0, idx1, acc_sh, den_sh,
                  semA, semB):
        cid = lax.axis_index("c")
        sid = lax.axis_index("s")
        base = (cid * NS + sid) * rows_per_w

        zeros16 = jnp.zeros((16,), jnp.float32)

        def zrow(r, _):
            for t in range(d // 16):
                rows0[r, pl.ds(t * 16, 16)] = zeros16
            return 0

        lax.fori_loop(0, chunk, zrow, 0)
        for t in range(chunk // 16):
            e0[pl.ds(t * 16, 16)] = zeros16
        pltpu.sync_copy(rows0.at[pl.ds(0, seg_per_sub)],
                        acc_sh.at[pl.ds(sid * seg_per_sub, seg_per_sub)])
        pltpu.sync_copy(e0.at[pl.ds(0, seg_per_sub)],
                        den_sh.at[pl.ds(sid * seg_per_sub, seg_per_sub)])
        plsc.subcore_barrier()

        def gather(c, rows, ev, idxv, sem):
            off = base + c * chunk
            pltpu.async_copy(wx_hbm.at[pl.ds(off, chunk)], rows, sem)
            pltpu.async_copy(e_hbm.at[pl.ds(off, chunk)], ev, sem)
            pltpu.async_copy(batch_hbm.at[pl.ds(off, chunk)], idxv, sem)

        def drain(rows, ev, idxv, sem):
            pltpu.make_async_copy(wx_hbm.at[pl.ds(0, chunk)], rows, sem).wait()
            pltpu.make_async_copy(e_hbm.at[pl.ds(0, chunk)], ev, sem).wait()
            pltpu.make_async_copy(batch_hbm.at[pl.ds(0, chunk)], idxv,
                                  sem).wait()

        def scatter(rows, ev, idxv):
            pltpu.sync_copy(rows, acc_sh.at[idxv], add=True)
            pltpu.sync_copy(ev, den_sh.at[idxv], add=True)

        gather(0, rows0, e0, idx0, semA)

        def body(kk, _):
            gather(2 * kk + 1, rows1, e1, idx1, semB)
            drain(rows0, e0, idx0, semA)
            scatter(rows0, e0, idx0)
            gather(2 * kk + 2, rows0, e0, idx0, semA)
            drain(rows1, e1, idx1, semB)
            scatter(rows1, e1, idx1)
            return 0

        lax.fori_loop(0, n_chunks // 2, body, 0)
        drain(rows0, e0, idx0, semA)
        scatter(rows0, e0, idx0)
        plsc.subcore_barrier()

        pltpu.sync_copy(
            acc_sh.at[pl.ds(sid * seg_per_sub, seg_per_sub)],
            acc_hbm.at[cid, pl.ds(sid * seg_per_sub, seg_per_sub)])
        pltpu.sync_copy(den_sh.at[pl.ds(sid * seg_per_sub, seg_per_sub)],
                        e0.at[pl.ds(0, seg_per_sub)])
        pltpu.sync_copy(e0.at[pl.ds(0, seg_per_sub)],
                        den_hbm.at[cid, pl.ds(sid * seg_per_sub, seg_per_sub)])

    return sc_kernel(wx, e, batch)


def _norm_body(acc0_ref, den0_ref, acc1_ref, den1_ref, o_ref):
    a = acc0_ref[0] + acc0_ref[1] + acc1_ref[0] + acc1_ref[1]
    dsum = den0_ref[0] + den0_ref[1] + den1_ref[0] + den1_ref[1]
    o_ref[...] = a / jnp.where(dsum > 0, dsum, 1.0)[:, None]


def _normalize(acc0, den0, acc1, den1, interpret=False):
    _, seg, d = acc0.shape
    return pl.pallas_call(
        _norm_body,
        out_shape=jax.ShapeDtypeStruct((seg, d), jnp.float32),
        interpret=interpret,
    )(acc0, den0, acc1, den1)


def kernel(node_features, batch, W1, b1, W2, b2):
    n = node_features.shape[0]
    half = n // 2
    wx0, e0 = _scores_premul(node_features, W1, b1, W2, b2,
                             block_rows=16000, row_start=0, row_count=half)
    acc0, den0 = _sc_scatter_call(wx0, e0, lax.slice(batch, (0,), (half,)),
                                  chunk=40)
    wx1, e1 = _scores_premul(node_features, W1, b1, W2, b2,
                             block_rows=16000, row_start=half, row_count=half)
    acc1, den1 = _sc_scatter_call(wx1, e1, lax.slice(batch, (half,), (n,)),
                                  chunk=40)
    return _normalize(acc0, den0, acc1, den1)


# speedup vs baseline: 1.2002x; 1.2002x over previous
import functools

import jax
import jax.numpy as jnp
from jax import lax
from jax.experimental import pallas as pl
from jax.experimental.pallas import tpu as pltpu
from jax.experimental.pallas import tpu_sc as plsc

SEG = 1024
NC = 2
NS = 16
NW = NC * NS


def _score_body(x_ref, w1_ref, b1_ref, w2_ref, w2r_ref, b2_ref,
                wx_ref, e_ref):
    x = x_ref[...]
    h = jnp.tanh(
        jax.lax.dot_general(x, w1_ref[...], (((1,), (0,)), ((), ())),
                            preferred_element_type=jnp.float32)
        + b1_ref[...])
    s = jax.lax.dot_general(h, w2_ref[...], (((1,), (0,)), ((), ())),
                            preferred_element_type=jnp.float32) + b2_ref[...]
    wx_ref[...] = x * jnp.exp(s)
    s_row = jax.lax.dot_general(w2r_ref[...], h, (((1,), (1,)), ((), ())),
                                preferred_element_type=jnp.float32)
    e_ref[...] = jnp.exp(s_row + b2_ref[...])[None]


def _scores_premul(x, w1, b1, w2, b2, block_rows, row_start=0,
                   row_count=None, interpret=False):
    n, d = x.shape
    nh = n if row_count is None else row_count
    grid = nh // block_rows
    blk0 = row_start // block_rows
    wx, e = pl.pallas_call(
        _score_body,
        grid=(grid,),
        in_specs=[
            pl.BlockSpec((block_rows, d), lambda i: (i + blk0, 0)),
            pl.BlockSpec((d, w1.shape[1]), lambda i: (0, 0)),
            pl.BlockSpec((1, w1.shape[1]), lambda i: (0, 0)),
            pl.BlockSpec((w1.shape[1], 1), lambda i: (0, 0)),
            pl.BlockSpec((1, w1.shape[1]), lambda i: (0, 0)),
            pl.BlockSpec((1, 1), lambda i: (0, 0)),
        ],
        out_specs=[
            pl.BlockSpec((block_rows, d), lambda i: (i, 0)),
            pl.BlockSpec((1, 1, block_rows), lambda i: (i, 0, 0)),
        ],
        out_shape=[
            jax.ShapeDtypeStruct((nh, d), jnp.float32),
            jax.ShapeDtypeStruct((grid, 1, block_rows), jnp.float32),
        ],
        interpret=interpret,
    )(x, w1, b1.reshape(1, -1), w2, w2.reshape(1, -1), b2.reshape(1, 1))
    return wx, e.reshape(nh)


def _sc_scatter_call(wx, e, batch, chunk):
    n, d = wx.shape
    rows_per_w = n // NW
    n_chunks = rows_per_w // chunk
    mesh = plsc.VectorSubcoreMesh(core_axis_name="c", subcore_axis_name="s")
    seg_per_sub = SEG // NS
    n_pairs = (n_chunks - 1) // 2

    @functools.partial(
        pl.kernel,
        out_type=[
            jax.ShapeDtypeStruct((NC, SEG, d), jnp.float32),
            jax.ShapeDtypeStruct((NC, SEG), jnp.float32),
        ],
        mesh=mesh,
        scratch_types=[
            pltpu.VMEM((chunk, d), jnp.float32),
            pltpu.VMEM((chunk, d), jnp.float32),
            pltpu.VMEM((chunk,), jnp.float32),
            pltpu.VMEM((chunk,), jnp.float32),
            pltpu.VMEM((chunk,), jnp.int32),
            pltpu.VMEM((chunk,), jnp.int32),
            pltpu.VMEM_SHARED((SEG, d), jnp.float32),
            pltpu.VMEM_SHARED((SEG,), jnp.float32),
            pltpu.SemaphoreType.DMA,
            pltpu.SemaphoreType.DMA,
        ],
    )
    def sc_kernel(wx_hbm, e_hbm, batch_hbm, acc_hbm, den_hbm,
                  rows0, rows1, e0, e1, idx0, idx1, acc_sh, den_sh,
                  semA, semB):
        cid = lax.axis_index("c")
        sid = lax.axis_index("s")
        base = (cid * NS + sid) * rows_per_w

        zeros16 = jnp.zeros((16,), jnp.float32)

        def zrow(r, _):
            for t in range(d // 16):
                rows0[r, pl.ds(t * 16, 16)] = zeros16
            return 0

        lax.fori_loop(0, chunk, zrow, 0)
        for t in range(chunk // 16):
            e0[pl.ds(t * 16, 16)] = zeros16
        pltpu.sync_copy(rows0.at[pl.ds(0, seg_per_sub)],
                        acc_sh.at[pl.ds(sid * seg_per_sub, seg_per_sub)])
        pltpu.sync_copy(e0.at[pl.ds(0, seg_per_sub)],
                        den_sh.at[pl.ds(sid * seg_per_sub, seg_per_sub)])
        plsc.subcore_barrier()

        def gather(c, rows, ev, idxv, sem):
            off = base + c * chunk
            pltpu.async_copy(wx_hbm.at[pl.ds(off, chunk)], rows, sem)
            pltpu.async_copy(e_hbm.at[pl.ds(off, chunk)], ev, sem)
            pltpu.async_copy(batch_hbm.at[pl.ds(off, chunk)], idxv, sem)

        def drain(rows, ev, idxv, sem):
            pltpu.make_async_copy(wx_hbm.at[pl.ds(0, chunk)], rows, sem).wait()
            pltpu.make_async_copy(e_hbm.at[pl.ds(0, chunk)], ev, sem).wait()
            pltpu.make_async_copy(batch_hbm.at[pl.ds(0, chunk)], idxv,
                                  sem).wait()

        def scatter(rows, ev, idxv):
            pltpu.sync_copy(rows, acc_sh.at[idxv], add=True)
            pltpu.sync_copy(ev, den_sh.at[idxv], add=True)

        gather(0, rows0, e0, idx0, semA)

        def body(kk, _):
            gather(2 * kk + 1, rows1, e1, idx1, semB)
            drain(rows0, e0, idx0, semA)
            scatter(rows0, e0, idx0)
            gather(2 * kk + 2, rows0, e0, idx0, semA)
            drain(rows1, e1, idx1, semB)
            scatter(rows1, e1, idx1)
            return 0

        lax.fori_loop(0, n_pairs, body, 0)
        if n_chunks % 2 == 0:
            gather(n_chunks - 1, rows1, e1, idx1, semB)
        drain(rows0, e0, idx0, semA)
        scatter(rows0, e0, idx0)
        if n_chunks % 2 == 0:
            drain(rows1, e1, idx1, semB)
            scatter(rows1, e1, idx1)
        plsc.subcore_barrier()

        pltpu.sync_copy(
            acc_sh.at[pl.ds(sid * seg_per_sub, seg_per_sub)],
            acc_hbm.at[cid, pl.ds(sid * seg_per_sub, seg_per_sub)])
        pltpu.sync_copy(den_sh.at[pl.ds(sid * seg_per_sub, seg_per_sub)],
                        e0.at[pl.ds(0, seg_per_sub)])
        pltpu.sync_copy(e0.at[pl.ds(0, seg_per_sub)],
                        den_hbm.at[cid, pl.ds(sid * seg_per_sub, seg_per_sub)])

    return sc_kernel(wx, e, batch)


def _norm_body(acc0_ref, den0_ref, acc1_ref, den1_ref, o_ref):
    a = acc0_ref[0] + acc0_ref[1] + acc1_ref[0] + acc1_ref[1]
    dsum = den0_ref[0] + den0_ref[1] + den1_ref[0] + den1_ref[1]
    o_ref[...] = a / jnp.where(dsum > 0, dsum, 1.0)[:, None]


def _normalize(acc0, den0, acc1, den1, interpret=False):
    _, seg, d = acc0.shape
    return pl.pallas_call(
        _norm_body,
        out_shape=jax.ShapeDtypeStruct((seg, d), jnp.float32),
        interpret=interpret,
    )(acc0, den0, acc1, den1)


def kernel(node_features, batch, W1, b1, W2, b2):
    n = node_features.shape[0]
    half = 166400
    wx0, e0 = _scores_premul(node_features, W1, b1, W2, b2,
                             block_rows=12800, row_start=0, row_count=half)
    acc0, den0 = _sc_scatter_call(wx0, e0, lax.slice(batch, (0,), (half,)),
                                  chunk=80)
    wx1, e1 = _scores_premul(node_features, W1, b1, W2, b2,
                             block_rows=12800, row_start=half,
                             row_count=n - half)
    acc1, den1 = _sc_scatter_call(wx1, e1, lax.slice(batch, (half,), (n,)),
                                  chunk=80)
    return _normalize(acc0, den0, acc1, den1)


# speedup vs baseline: 1.2194x; 1.0160x over previous
import functools

import jax
import jax.numpy as jnp
from jax import lax
from jax.experimental import pallas as pl
from jax.experimental.pallas import tpu as pltpu
from jax.experimental.pallas import tpu_sc as plsc

SEG = 1024
NC = 2
NS = 16
NW = NC * NS


def _score_body(x_ref, w1_ref, b1_ref, w2_ref, w2r_ref, b2_ref,
                wx_ref, e_ref):
    x = x_ref[...]
    h = jnp.tanh(
        jax.lax.dot_general(x, w1_ref[...], (((1,), (0,)), ((), ())),
                            preferred_element_type=jnp.float32)
        + b1_ref[...])
    s = jax.lax.dot_general(h, w2_ref[...], (((1,), (0,)), ((), ())),
                            preferred_element_type=jnp.float32) + b2_ref[...]
    wx_ref[...] = x * jnp.exp(s)
    s_row = jax.lax.dot_general(w2r_ref[...], h, (((1,), (1,)), ((), ())),
                                preferred_element_type=jnp.float32)
    e_ref[...] = jnp.exp(s_row + b2_ref[...])[None]


def _scores_premul(x, w1, b1, w2, b2, block_rows, row_start=0,
                   row_count=None, interpret=False):
    n, d = x.shape
    nh = n if row_count is None else row_count
    grid = nh // block_rows
    blk0 = row_start // block_rows
    wx, e = pl.pallas_call(
        _score_body,
        grid=(grid,),
        in_specs=[
            pl.BlockSpec((block_rows, d), lambda i: (i + blk0, 0)),
            pl.BlockSpec((d, w1.shape[1]), lambda i: (0, 0)),
            pl.BlockSpec((1, w1.shape[1]), lambda i: (0, 0)),
            pl.BlockSpec((w1.shape[1], 1), lambda i: (0, 0)),
            pl.BlockSpec((1, w1.shape[1]), lambda i: (0, 0)),
            pl.BlockSpec((1, 1), lambda i: (0, 0)),
        ],
        out_specs=[
            pl.BlockSpec((block_rows, d), lambda i: (i, 0)),
            pl.BlockSpec((1, 1, block_rows), lambda i: (i, 0, 0)),
        ],
        out_shape=[
            jax.ShapeDtypeStruct((nh, d), jnp.float32),
            jax.ShapeDtypeStruct((grid, 1, block_rows), jnp.float32),
        ],
        interpret=interpret,
    )(x, w1, b1.reshape(1, -1), w2, w2.reshape(1, -1), b2.reshape(1, 1))
    return wx, e.reshape(nh)


def _sc_scatter_call(wx, e, batch, chunk):
    n, d = wx.shape
    rows_per_w = n // NW
    n_chunks = rows_per_w // chunk
    mesh = plsc.VectorSubcoreMesh(core_axis_name="c", subcore_axis_name="s")
    seg_per_sub = SEG // NS
    n_pairs = (n_chunks - 1) // 2

    @functools.partial(
        pl.kernel,
        out_type=[
            jax.ShapeDtypeStruct((NC, SEG, d), jnp.float32),
            jax.ShapeDtypeStruct((NC, SEG), jnp.float32),
        ],
        mesh=mesh,
        scratch_types=[
            pltpu.VMEM((chunk, d), jnp.float32),
            pltpu.VMEM((chunk, d), jnp.float32),
            pltpu.VMEM((chunk,), jnp.float32),
            pltpu.VMEM((chunk,), jnp.float32),
            pltpu.VMEM((chunk,), jnp.int32),
            pltpu.VMEM((chunk,), jnp.int32),
            pltpu.VMEM_SHARED((SEG, d), jnp.float32),
            pltpu.VMEM_SHARED((SEG,), jnp.float32),
            pltpu.SemaphoreType.DMA,
            pltpu.SemaphoreType.DMA,
        ],
    )
    def sc_kernel(wx_hbm, e_hbm, batch_hbm, acc_hbm, den_hbm,
                  rows0, rows1, e0, e1, idx0, idx1, acc_sh, den_sh,
                  semA, semB):
        cid = lax.axis_index("c")
        sid = lax.axis_index("s")
        base = (cid * NS + sid) * rows_per_w

        zeros16 = jnp.zeros((16,), jnp.float32)

        def zrow(r, _):
            for t in range(d // 16):
                rows0[r, pl.ds(t * 16, 16)] = zeros16
            return 0

        lax.fori_loop(0, chunk, zrow, 0)
        for t in range(chunk // 16):
            e0[pl.ds(t * 16, 16)] = zeros16
        pltpu.sync_copy(rows0.at[pl.ds(0, seg_per_sub)],
                        acc_sh.at[pl.ds(sid * seg_per_sub, seg_per_sub)])
        pltpu.sync_copy(e0.at[pl.ds(0, seg_per_sub)],
                        den_sh.at[pl.ds(sid * seg_per_sub, seg_per_sub)])
        plsc.subcore_barrier()

        def gather(c, rows, ev, idxv, sem):
            off = base + c * chunk
            pltpu.async_copy(wx_hbm.at[pl.ds(off, chunk)], rows, sem)
            pltpu.async_copy(e_hbm.at[pl.ds(off, chunk)], ev, sem)
            pltpu.async_copy(batch_hbm.at[pl.ds(off, chunk)], idxv, sem)

        def drain(rows, ev, idxv, sem):
            pltpu.make_async_copy(wx_hbm.at[pl.ds(0, chunk)], rows, sem).wait()
            pltpu.make_async_copy(e_hbm.at[pl.ds(0, chunk)], ev, sem).wait()
            pltpu.make_async_copy(batch_hbm.at[pl.ds(0, chunk)], idxv,
                                  sem).wait()

        def scatter(rows, ev, idxv):
            pltpu.sync_copy(rows, acc_sh.at[idxv], add=True)
            pltpu.sync_copy(ev, den_sh.at[idxv], add=True)

        gather(0, rows0, e0, idx0, semA)

        def body(kk, _):
            gather(2 * kk + 1, rows1, e1, idx1, semB)
            drain(rows0, e0, idx0, semA)
            scatter(rows0, e0, idx0)
            gather(2 * kk + 2, rows0, e0, idx0, semA)
            drain(rows1, e1, idx1, semB)
            scatter(rows1, e1, idx1)
            return 0

        lax.fori_loop(0, n_pairs, body, 0)
        if n_chunks % 2 == 0:
            gather(n_chunks - 1, rows1, e1, idx1, semB)
        drain(rows0, e0, idx0, semA)
        scatter(rows0, e0, idx0)
        if n_chunks % 2 == 0:
            drain(rows1, e1, idx1, semB)
            scatter(rows1, e1, idx1)
        plsc.subcore_barrier()

        pltpu.sync_copy(
            acc_sh.at[pl.ds(sid * seg_per_sub, seg_per_sub)],
            acc_hbm.at[cid, pl.ds(sid * seg_per_sub, seg_per_sub)])
        pltpu.sync_copy(den_sh.at[pl.ds(sid * seg_per_sub, seg_per_sub)],
                        e0.at[pl.ds(0, seg_per_sub)])
        pltpu.sync_copy(e0.at[pl.ds(0, seg_per_sub)],
                        den_hbm.at[cid, pl.ds(sid * seg_per_sub, seg_per_sub)])

    return sc_kernel(wx, e, batch)


def _norm_body(*refs):
    o_ref = refs[-1]
    nparts = (len(refs) - 1) // 2
    acc_refs = refs[:nparts]
    den_refs = refs[nparts:-1]
    a = sum(r[0] + r[1] for r in acc_refs)
    dsum = sum(r[0] + r[1] for r in den_refs)
    o_ref[...] = a / jnp.where(dsum > 0, dsum, 1.0)[:, None]


def _normalize(accs, dens, interpret=False):
    _, seg, d = accs[0].shape
    return pl.pallas_call(
        _norm_body,
        out_shape=jax.ShapeDtypeStruct((seg, d), jnp.float32),
        interpret=interpret,
    )(*accs, *dens)


def kernel(node_features, batch, W1, b1, W2, b2):
    n = node_features.shape[0]
    bounds = [0, 76800, 153600, 230400, n]
    parts = []
    for lo, hi in zip(bounds[:-1], bounds[1:]):
        wx_p, e_p = _scores_premul(node_features, W1, b1, W2, b2,
                                   block_rows=12800, row_start=lo,
                                   row_count=hi - lo)
        parts.append(_sc_scatter_call(
            wx_p, e_p, lax.slice(batch, (lo,), (hi,)), chunk=80))
    accs = [p[0] for p in parts]
    dens = [p[1] for p in parts]
    return _normalize(accs, dens)


# speedup vs baseline: 1.2300x; 1.0087x over previous
import functools

import jax
import jax.numpy as jnp
from jax import lax
from jax.experimental import pallas as pl
from jax.experimental.pallas import tpu as pltpu
from jax.experimental.pallas import tpu_sc as plsc

SEG = 1024
NC = 2
NS = 16
NW = NC * NS


def _score_body(x_ref, w1_ref, b1_ref, w2_ref, w2r_ref, b2_ref,
                wx_ref, e_ref):
    x = x_ref[...]
    h = jnp.tanh(
        jax.lax.dot_general(x, w1_ref[...], (((1,), (0,)), ((), ())),
                            preferred_element_type=jnp.float32)
        + b1_ref[...])
    s = jax.lax.dot_general(h, w2_ref[...], (((1,), (0,)), ((), ())),
                            preferred_element_type=jnp.float32) + b2_ref[...]
    wx_ref[...] = x * jnp.exp(s)
    s_row = jax.lax.dot_general(w2r_ref[...], h, (((1,), (1,)), ((), ())),
                                preferred_element_type=jnp.float32)
    e_ref[...] = jnp.exp(s_row + b2_ref[...])[None]


def _scores_premul(x, w1, b1, w2, b2, block_rows, row_start=0,
                   row_count=None, interpret=False):
    n, d = x.shape
    nh = n if row_count is None else row_count
    grid = nh // block_rows
    blk0 = row_start // block_rows
    wx, e = pl.pallas_call(
        _score_body,
        grid=(grid,),
        in_specs=[
            pl.BlockSpec((block_rows, d), lambda i: (i + blk0, 0)),
            pl.BlockSpec((d, w1.shape[1]), lambda i: (0, 0)),
            pl.BlockSpec((1, w1.shape[1]), lambda i: (0, 0)),
            pl.BlockSpec((w1.shape[1], 1), lambda i: (0, 0)),
            pl.BlockSpec((1, w1.shape[1]), lambda i: (0, 0)),
            pl.BlockSpec((1, 1), lambda i: (0, 0)),
        ],
        out_specs=[
            pl.BlockSpec((block_rows, d), lambda i: (i, 0)),
            pl.BlockSpec((1, 1, block_rows), lambda i: (i, 0, 0)),
        ],
        out_shape=[
            jax.ShapeDtypeStruct((nh, d), jnp.float32),
            jax.ShapeDtypeStruct((grid, 1, block_rows), jnp.float32),
        ],
        interpret=interpret,
    )(x, w1, b1.reshape(1, -1), w2, w2.reshape(1, -1), b2.reshape(1, 1))
    return wx, e.reshape(nh)


def _sc_scatter_call(wx, e, batch, chunk):
    n, d = wx.shape
    rows_per_w = n // NW
    n_chunks = rows_per_w // chunk
    mesh = plsc.VectorSubcoreMesh(core_axis_name="c", subcore_axis_name="s")
    seg_per_sub = SEG // NS
    n_pairs = (n_chunks - 1) // 2

    @functools.partial(
        pl.kernel,
        out_type=[
            jax.ShapeDtypeStruct((NC, SEG, d), jnp.float32),
            jax.ShapeDtypeStruct((NC, SEG), jnp.float32),
        ],
        mesh=mesh,
        scratch_types=[
            pltpu.VMEM((chunk, d), jnp.float32),
            pltpu.VMEM((chunk, d), jnp.float32),
            pltpu.VMEM((chunk,), jnp.float32),
            pltpu.VMEM((chunk,), jnp.float32),
            pltpu.VMEM((chunk,), jnp.int32),
            pltpu.VMEM((chunk,), jnp.int32),
            pltpu.VMEM_SHARED((SEG, d), jnp.float32),
            pltpu.VMEM_SHARED((SEG,), jnp.float32),
            pltpu.SemaphoreType.DMA,
            pltpu.SemaphoreType.DMA,
        ],
    )
    def sc_kernel(wx_hbm, e_hbm, batch_hbm, acc_hbm, den_hbm,
                  rows0, rows1, e0, e1, idx0, idx1, acc_sh, den_sh,
                  semA, semB):
        cid = lax.axis_index("c")
        sid = lax.axis_index("s")
        base = (cid * NS + sid) * rows_per_w

        zeros16 = jnp.zeros((16,), jnp.float32)

        def zrow(r, _):
            for t in range(d // 16):
                rows0[r, pl.ds(t * 16, 16)] = zeros16
            return 0

        lax.fori_loop(0, chunk, zrow, 0)
        for t in range(chunk // 16):
            e0[pl.ds(t * 16, 16)] = zeros16
        pltpu.sync_copy(rows0.at[pl.ds(0, seg_per_sub)],
                        acc_sh.at[pl.ds(sid * seg_per_sub, seg_per_sub)])
        pltpu.sync_copy(e0.at[pl.ds(0, seg_per_sub)],
                        den_sh.at[pl.ds(sid * seg_per_sub, seg_per_sub)])
        plsc.subcore_barrier()

        def gather(c, rows, ev, idxv, sem):
            off = base + c * chunk
            pltpu.async_copy(wx_hbm.at[pl.ds(off, chunk)], rows, sem)
            pltpu.async_copy(e_hbm.at[pl.ds(off, chunk)], ev, sem)
            pltpu.async_copy(batch_hbm.at[pl.ds(off, chunk)], idxv, sem)

        def drain(rows, ev, idxv, sem):
            pltpu.make_async_copy(wx_hbm.at[pl.ds(0, chunk)], rows, sem).wait()
            pltpu.make_async_copy(e_hbm.at[pl.ds(0, chunk)], ev, sem).wait()
            pltpu.make_async_copy(batch_hbm.at[pl.ds(0, chunk)], idxv,
                                  sem).wait()

        def scatter(rows, ev, idxv):
            pltpu.sync_copy(rows, acc_sh.at[idxv], add=True)
            pltpu.sync_copy(ev, den_sh.at[idxv], add=True)

        gather(0, rows0, e0, idx0, semA)

        def body(kk, _):
            gather(2 * kk + 1, rows1, e1, idx1, semB)
            drain(rows0, e0, idx0, semA)
            scatter(rows0, e0, idx0)
            gather(2 * kk + 2, rows0, e0, idx0, semA)
            drain(rows1, e1, idx1, semB)
            scatter(rows1, e1, idx1)
            return 0

        lax.fori_loop(0, n_pairs, body, 0)
        if n_chunks % 2 == 0:
            gather(n_chunks - 1, rows1, e1, idx1, semB)
        drain(rows0, e0, idx0, semA)
        scatter(rows0, e0, idx0)
        if n_chunks % 2 == 0:
            drain(rows1, e1, idx1, semB)
            scatter(rows1, e1, idx1)
        plsc.subcore_barrier()

        pltpu.sync_copy(
            acc_sh.at[pl.ds(sid * seg_per_sub, seg_per_sub)],
            acc_hbm.at[cid, pl.ds(sid * seg_per_sub, seg_per_sub)])
        pltpu.sync_copy(den_sh.at[pl.ds(sid * seg_per_sub, seg_per_sub)],
                        e0.at[pl.ds(0, seg_per_sub)])
        pltpu.sync_copy(e0.at[pl.ds(0, seg_per_sub)],
                        den_hbm.at[cid, pl.ds(sid * seg_per_sub, seg_per_sub)])

    return sc_kernel(wx, e, batch)


def _norm_body(*refs):
    o_ref = refs[-1]
    nparts = (len(refs) - 1) // 2
    acc_refs = refs[:nparts]
    den_refs = refs[nparts:-1]
    a = sum(r[0] + r[1] for r in acc_refs)
    dsum = sum(r[0] + r[1] for r in den_refs)
    o_ref[...] = a / jnp.where(dsum > 0, dsum, 1.0)[:, None]


def _normalize(accs, dens, interpret=False):
    _, seg, d = accs[0].shape
    return pl.pallas_call(
        _norm_body,
        out_shape=jax.ShapeDtypeStruct((seg, d), jnp.float32),
        interpret=interpret,
    )(*accs, *dens)


def kernel(node_features, batch, W1, b1, W2, b2):
    n = node_features.shape[0]
    bounds = [0, 102400, 204800, n]
    parts = []
    for lo, hi in zip(bounds[:-1], bounds[1:]):
        wx_p, e_p = _scores_premul(node_features, W1, b1, W2, b2,
                                   block_rows=12800, row_start=lo,
                                   row_count=hi - lo)
        parts.append(_sc_scatter_call(
            wx_p, e_p, lax.slice(batch, (lo,), (hi,)), chunk=80))
    accs = [p[0] for p in parts]
    dens = [p[1] for p in parts]
    return _normalize(accs, dens)


# speedup vs baseline: 1.2362x; 1.0051x over previous
import functools

import jax
import jax.numpy as jnp
from jax import lax
from jax.experimental import pallas as pl
from jax.experimental.pallas import tpu as pltpu
from jax.experimental.pallas import tpu_sc as plsc

SEG = 1024
NC = 2
NS = 16
NW = NC * NS


def _score_body(x_ref, w1_ref, b1_ref, w2_ref, w2r_ref, b2_ref,
                wx_ref, e_ref):
    x = x_ref[...]
    h = jnp.tanh(
        jax.lax.dot_general(x, w1_ref[...], (((1,), (0,)), ((), ())),
                            preferred_element_type=jnp.float32)
        + b1_ref[...])
    s = jax.lax.dot_general(h, w2_ref[...], (((1,), (0,)), ((), ())),
                            preferred_element_type=jnp.float32) + b2_ref[...]
    wx_ref[...] = x * jnp.exp(s)
    s_row = jax.lax.dot_general(w2r_ref[...], h, (((1,), (1,)), ((), ())),
                                preferred_element_type=jnp.float32)
    e_ref[...] = jnp.exp(s_row + b2_ref[...])[None]


def _scores_premul(x, w1, b1, w2, b2, block_rows, row_start=0,
                   row_count=None, interpret=False):
    n, d = x.shape
    nh = n if row_count is None else row_count
    grid = nh // block_rows
    blk0 = row_start // block_rows
    wx, e = pl.pallas_call(
        _score_body,
        grid=(grid,),
        in_specs=[
            pl.BlockSpec((block_rows, d), lambda i: (i + blk0, 0)),
            pl.BlockSpec((d, w1.shape[1]), lambda i: (0, 0)),
            pl.BlockSpec((1, w1.shape[1]), lambda i: (0, 0)),
            pl.BlockSpec((w1.shape[1], 1), lambda i: (0, 0)),
            pl.BlockSpec((1, w1.shape[1]), lambda i: (0, 0)),
            pl.BlockSpec((1, 1), lambda i: (0, 0)),
        ],
        out_specs=[
            pl.BlockSpec((block_rows, d), lambda i: (i, 0)),
            pl.BlockSpec((1, 1, block_rows), lambda i: (i, 0, 0)),
        ],
        out_shape=[
            jax.ShapeDtypeStruct((nh, d), jnp.float32),
            jax.ShapeDtypeStruct((grid, 1, block_rows), jnp.float32),
        ],
        interpret=interpret,
    )(x, w1, b1.reshape(1, -1), w2, w2.reshape(1, -1), b2.reshape(1, 1))
    return wx, e.reshape(nh)


def _sc_scatter_call(wx, e, batch, chunk):
    n, d = wx.shape
    rows_per_w = n // NW
    n_chunks = rows_per_w // chunk
    mesh = plsc.VectorSubcoreMesh(core_axis_name="c", subcore_axis_name="s")
    seg_per_sub = SEG // NS
    n_pairs = (n_chunks - 1) // 2

    @functools.partial(
        pl.kernel,
        out_type=[
            jax.ShapeDtypeStruct((NC, SEG, d), jnp.float32),
            jax.ShapeDtypeStruct((NC, SEG), jnp.float32),
        ],
        mesh=mesh,
        scratch_types=[
            pltpu.VMEM((chunk, d), jnp.float32),
            pltpu.VMEM((chunk, d), jnp.float32),
            pltpu.VMEM((chunk,), jnp.float32),
            pltpu.VMEM((chunk,), jnp.float32),
            pltpu.VMEM((chunk,), jnp.int32),
            pltpu.VMEM((chunk,), jnp.int32),
            pltpu.VMEM_SHARED((SEG, d), jnp.float32),
            pltpu.VMEM_SHARED((SEG,), jnp.float32),
            pltpu.SemaphoreType.DMA,
            pltpu.SemaphoreType.DMA,
            pltpu.SemaphoreType.DMA,
            pltpu.SemaphoreType.DMA,
        ],
    )
    def sc_kernel(wx_hbm, e_hbm, batch_hbm, acc_hbm, den_hbm,
                  rows0, rows1, e0, e1, idx0, idx1, acc_sh, den_sh,
                  semA, semB, semDA, semDB):
        cid = lax.axis_index("c")
        sid = lax.axis_index("s")
        base = (cid * NS + sid) * rows_per_w

        zeros16 = jnp.zeros((16,), jnp.float32)

        def zrow(r, _):
            for t in range(d // 16):
                rows0[r, pl.ds(t * 16, 16)] = zeros16
            return 0

        lax.fori_loop(0, chunk, zrow, 0)
        for t in range(chunk // 16):
            e0[pl.ds(t * 16, 16)] = zeros16
        pltpu.sync_copy(rows0.at[pl.ds(0, seg_per_sub)],
                        acc_sh.at[pl.ds(sid * seg_per_sub, seg_per_sub)])
        pltpu.sync_copy(e0.at[pl.ds(0, seg_per_sub)],
                        den_sh.at[pl.ds(sid * seg_per_sub, seg_per_sub)])
        plsc.subcore_barrier()

        def gather(c, rows, ev, idxv, sem):
            off = base + c * chunk
            pltpu.async_copy(wx_hbm.at[pl.ds(off, chunk)], rows, sem)
            pltpu.async_copy(e_hbm.at[pl.ds(off, chunk)], ev, sem)
            pltpu.async_copy(batch_hbm.at[pl.ds(off, chunk)], idxv, sem)

        def drain(rows, ev, idxv, sem):
            pltpu.make_async_copy(wx_hbm.at[pl.ds(0, chunk)], rows, sem).wait()
            pltpu.make_async_copy(e_hbm.at[pl.ds(0, chunk)], ev, sem).wait()
            pltpu.make_async_copy(batch_hbm.at[pl.ds(0, chunk)], idxv,
                                  sem).wait()

        def scatter(rows, ev, idxv, dsem):
            pltpu.async_copy(ev, den_sh.at[idxv], dsem, add=True)
            pltpu.sync_copy(rows, acc_sh.at[idxv], add=True)
            pltpu.make_async_copy(ev, den_sh.at[idxv], dsem).wait()

        gather(0, rows0, e0, idx0, semA)

        def body(kk, _):
            gather(2 * kk + 1, rows1, e1, idx1, semB)
            drain(rows0, e0, idx0, semA)
            scatter(rows0, e0, idx0, semDA)
            gather(2 * kk + 2, rows0, e0, idx0, semA)
            drain(rows1, e1, idx1, semB)
            scatter(rows1, e1, idx1, semDB)
            return 0

        lax.fori_loop(0, n_pairs, body, 0)
        if n_chunks % 2 == 0:
            gather(n_chunks - 1, rows1, e1, idx1, semB)
        drain(rows0, e0, idx0, semA)
        scatter(rows0, e0, idx0, semDA)
        if n_chunks % 2 == 0:
            drain(rows1, e1, idx1, semB)
            scatter(rows1, e1, idx1, semDB)
        plsc.subcore_barrier()

        pltpu.sync_copy(
            acc_sh.at[pl.ds(sid * seg_per_sub, seg_per_sub)],
            acc_hbm.at[cid, pl.ds(sid * seg_per_sub, seg_per_sub)])
        pltpu.sync_copy(den_sh.at[pl.ds(sid * seg_per_sub, seg_per_sub)],
                        e0.at[pl.ds(0, seg_per_sub)])
        pltpu.sync_copy(e0.at[pl.ds(0, seg_per_sub)],
                        den_hbm.at[cid, pl.ds(sid * seg_per_sub, seg_per_sub)])

    return sc_kernel(wx, e, batch)


def _norm_body(*refs):
    o_ref = refs[-1]
    nparts = (len(refs) - 1) // 2
    acc_refs = refs[:nparts]
    den_refs = refs[nparts:-1]
    a = sum(r[0] + r[1] for r in acc_refs)
    dsum = sum(r[0] + r[1] for r in den_refs)
    o_ref[...] = a / jnp.where(dsum > 0, dsum, 1.0)[:, None]


def _normalize(accs, dens, interpret=False):
    _, seg, d = accs[0].shape
    return pl.pallas_call(
        _norm_body,
        out_shape=jax.ShapeDtypeStruct((seg, d), jnp.float32),
        interpret=interpret,
    )(*accs, *dens)


def kernel(node_features, batch, W1, b1, W2, b2):
    n = node_features.shape[0]
    bounds = [0, 102400, 204800, n]
    parts = []
    for lo, hi in zip(bounds[:-1], bounds[1:]):
        wx_p, e_p = _scores_premul(node_features, W1, b1, W2, b2,
                                   block_rows=12800, row_start=lo,
                                   row_count=hi - lo)
        parts.append(_sc_scatter_call(
            wx_p, e_p, lax.slice(batch, (lo,), (hi,)), chunk=80))
    accs = [p[0] for p in parts]
    dens = [p[1] for p in parts]
    return _normalize(accs, dens)


# speedup vs baseline: 1.2614x; 1.0204x over previous
import functools

import jax
import jax.numpy as jnp
from jax import lax
from jax.experimental import pallas as pl
from jax.experimental.pallas import tpu as pltpu
from jax.experimental.pallas import tpu_sc as plsc

SEG = 1024
NC = 2
NS = 16
NW = NC * NS


def _score_body(x_ref, w1_ref, b1_ref, w2_ref, w2r_ref, b2_ref,
                wx_ref, e_ref):
    x = x_ref[...]
    h = jnp.tanh(
        jax.lax.dot_general(x, w1_ref[...], (((1,), (0,)), ((), ())),
                            preferred_element_type=jnp.float32)
        + b1_ref[...])
    s = jax.lax.dot_general(h, w2_ref[...], (((1,), (0,)), ((), ())),
                            preferred_element_type=jnp.float32) + b2_ref[...]
    wx_ref[...] = x * jnp.exp(s)
    s_row = jax.lax.dot_general(w2r_ref[...], h, (((1,), (1,)), ((), ())),
                                preferred_element_type=jnp.float32)
    e_ref[...] = jnp.exp(s_row + b2_ref[...])[None]


def _scores_premul(x, w1, b1, w2, b2, block_rows, row_start=0,
                   row_count=None, interpret=False):
    n, d = x.shape
    nh = n if row_count is None else row_count
    grid = nh // block_rows
    blk0 = row_start // block_rows
    wx, e = pl.pallas_call(
        _score_body,
        grid=(grid,),
        in_specs=[
            pl.BlockSpec((block_rows, d), lambda i: (i + blk0, 0)),
            pl.BlockSpec((d, w1.shape[1]), lambda i: (0, 0)),
            pl.BlockSpec((1, w1.shape[1]), lambda i: (0, 0)),
            pl.BlockSpec((w1.shape[1], 1), lambda i: (0, 0)),
            pl.BlockSpec((1, w1.shape[1]), lambda i: (0, 0)),
            pl.BlockSpec((1, 1), lambda i: (0, 0)),
        ],
        out_specs=[
            pl.BlockSpec((block_rows, d), lambda i: (i, 0)),
            pl.BlockSpec((1, 1, block_rows), lambda i: (i, 0, 0)),
        ],
        out_shape=[
            jax.ShapeDtypeStruct((nh, d), jnp.float32),
            jax.ShapeDtypeStruct((grid, 1, block_rows), jnp.float32),
        ],
        interpret=interpret,
    )(x, w1, b1.reshape(1, -1), w2, w2.reshape(1, -1), b2.reshape(1, 1))
    return wx, e.reshape(nh)


def _sc_scatter_call(wx, e, batch, chunk):
    n, d = wx.shape
    rows_per_w = n // NW
    n_chunks = rows_per_w // chunk
    mesh = plsc.VectorSubcoreMesh(core_axis_name="c", subcore_axis_name="s")
    seg_per_sub = SEG // NS
    n_pairs = (n_chunks - 1) // 2

    @functools.partial(
        pl.kernel,
        out_type=[
            jax.ShapeDtypeStruct((NC, SEG, d), jnp.float32),
            jax.ShapeDtypeStruct((NC, SEG), jnp.float32),
        ],
        mesh=mesh,
        scratch_types=[
            pltpu.VMEM((chunk, d), jnp.float32),
            pltpu.VMEM((chunk, d), jnp.float32),
            pltpu.VMEM((chunk,), jnp.float32),
            pltpu.VMEM((chunk,), jnp.float32),
            pltpu.VMEM((chunk,), jnp.int32),
            pltpu.VMEM((chunk,), jnp.int32),
            pltpu.VMEM_SHARED((SEG, d), jnp.float32),
            pltpu.VMEM_SHARED((SEG,), jnp.float32),
            pltpu.SemaphoreType.DMA,
            pltpu.SemaphoreType.DMA,
            pltpu.SemaphoreType.DMA,
            pltpu.SemaphoreType.DMA,
        ],
    )
    def sc_kernel(wx_hbm, e_hbm, batch_hbm, acc_hbm, den_hbm,
                  rows0, rows1, e0, e1, idx0, idx1, acc_sh, den_sh,
                  semA, semB, semDA, semDB):
        cid = lax.axis_index("c")
        sid = lax.axis_index("s")
        base = (cid * NS + sid) * rows_per_w

        zeros16 = jnp.zeros((16,), jnp.float32)

        def zrow(r, _):
            for t in range(d // 16):
                rows0[r, pl.ds(t * 16, 16)] = zeros16
            return 0

        lax.fori_loop(0, chunk, zrow, 0)
        for t in range(chunk // 16):
            e0[pl.ds(t * 16, 16)] = zeros16
        pltpu.sync_copy(rows0.at[pl.ds(0, seg_per_sub)],
                        acc_sh.at[pl.ds(sid * seg_per_sub, seg_per_sub)])
        pltpu.sync_copy(e0.at[pl.ds(0, seg_per_sub)],
                        den_sh.at[pl.ds(sid * seg_per_sub, seg_per_sub)])
        plsc.subcore_barrier()

        def gather(c, rows, ev, idxv, sem):
            off = base + c * chunk
            pltpu.async_copy(wx_hbm.at[pl.ds(off, chunk)], rows, sem)
            pltpu.async_copy(e_hbm.at[pl.ds(off, chunk)], ev, sem)
            pltpu.async_copy(batch_hbm.at[pl.ds(off, chunk)], idxv, sem)

        def drain(rows, ev, idxv, sem):
            pltpu.make_async_copy(wx_hbm.at[pl.ds(0, chunk)], rows, sem).wait()
            pltpu.make_async_copy(e_hbm.at[pl.ds(0, chunk)], ev, sem).wait()
            pltpu.make_async_copy(batch_hbm.at[pl.ds(0, chunk)], idxv,
                                  sem).wait()

        def scatter(rows, ev, idxv, dsem):
            pltpu.async_copy(ev, den_sh.at[idxv], dsem, add=True)
            pltpu.sync_copy(rows, acc_sh.at[idxv], add=True)
            pltpu.make_async_copy(ev, den_sh.at[idxv], dsem).wait()

        gather(0, rows0, e0, idx0, semA)

        def body(kk, _):
            gather(2 * kk + 1, rows1, e1, idx1, semB)
            drain(rows0, e0, idx0, semA)
            scatter(rows0, e0, idx0, semDA)
            gather(2 * kk + 2, rows0, e0, idx0, semA)
            drain(rows1, e1, idx1, semB)
            scatter(rows1, e1, idx1, semDB)
            return 0

        lax.fori_loop(0, n_pairs, body, 0)
        if n_chunks % 2 == 0:
            gather(n_chunks - 1, rows1, e1, idx1, semB)
        drain(rows0, e0, idx0, semA)
        scatter(rows0, e0, idx0, semDA)
        if n_chunks % 2 == 0:
            drain(rows1, e1, idx1, semB)
            scatter(rows1, e1, idx1, semDB)
        plsc.subcore_barrier()

        pltpu.sync_copy(
            acc_sh.at[pl.ds(sid * seg_per_sub, seg_per_sub)],
            acc_hbm.at[cid, pl.ds(sid * seg_per_sub, seg_per_sub)])
        pltpu.sync_copy(den_sh.at[pl.ds(sid * seg_per_sub, seg_per_sub)],
                        e0.at[pl.ds(0, seg_per_sub)])
        pltpu.sync_copy(e0.at[pl.ds(0, seg_per_sub)],
                        den_hbm.at[cid, pl.ds(sid * seg_per_sub, seg_per_sub)])

    return sc_kernel(wx, e, batch)


def _norm_body(*refs):
    o_ref = refs[-1]
    nparts = (len(refs) - 1) // 2
    acc_refs = refs[:nparts]
    den_refs = refs[nparts:-1]
    a = sum(r[0] + r[1] for r in acc_refs)
    dsum = sum(r[0] + r[1] for r in den_refs)
    o_ref[...] = a / jnp.where(dsum > 0, dsum, 1.0)[:, None]


def _normalize(accs, dens, interpret=False):
    _, seg, d = accs[0].shape
    return pl.pallas_call(
        _norm_body,
        out_shape=jax.ShapeDtypeStruct((seg, d), jnp.float32),
        interpret=interpret,
    )(*accs, *dens)


def kernel(node_features, batch, W1, b1, W2, b2):
    n = node_features.shape[0]
    bounds = [0, 102400, 204800, n]
    chunks = [128, 128, 80]
    parts = []
    for lo, hi, ck in zip(bounds[:-1], bounds[1:], chunks):
        wx_p, e_p = _scores_premul(node_features, W1, b1, W2, b2,
                                   block_rows=12800, row_start=lo,
                                   row_count=hi - lo)
        parts.append(_sc_scatter_call(
            wx_p, e_p, lax.slice(batch, (lo,), (hi,)), chunk=ck))
    accs = [p[0] for p in parts]
    dens = [p[1] for p in parts]
    return _normalize(accs, dens)
